# baseline (speedup 1.0000x reference)
import functools

import numpy as np
import jax
import jax.numpy as jnp
from jax import lax
from jax.experimental import pallas as pl
from jax.experimental.pallas import tpu as pltpu
from jax.experimental.pallas import tpu_sc as plsc

_NUM_SPH = 7
_NUM_RAD = 6
_CUTOFF = 5.0


def _sph_jl_np(l, x):
    x = np.asarray(x, dtype=np.float64)
    j0 = np.sin(x) / x
    if l == 0:
        return j0
    j1 = np.sin(x) / x**2 - np.cos(x) / x
    if l == 1:
        return j1
    jm, jc = j0, j1
    for i in range(1, l):
        jm, jc = jc, (2 * i + 1) / x * jc - jm
    return jc


def _sph_zeros(n, k):
    m = n + k
    zeros = [np.arange(1, m + 1) * np.pi]
    for l in range(1, n):
        prev = zeros[-1]
        cur = []
        for j in range(len(prev) - 1):
            a, b = float(prev[j]), float(prev[j + 1])
            fa = float(_sph_jl_np(l, a))
            for _ in range(100):
                c = 0.5 * (a + b)
                fc = float(_sph_jl_np(l, c))
                if fa * fc <= 0.0:
                    b = c
                else:
                    a, fa = c, fc
            cur.append(0.5 * (a + b))
        zeros.append(np.asarray(cur))
    return np.stack([z[:k] for z in zeros], axis=0)


_Z = _sph_zeros(_NUM_SPH, _NUM_RAD)
_NORMC = np.zeros((_NUM_SPH, _NUM_RAD))
for _l in range(_NUM_SPH):
    for _j in range(_NUM_RAD):
        _NORMC[_l, _j] = 1.0 / np.sqrt(0.5 * _sph_jl_np(_l + 1, _Z[_l, _j]) ** 2)
_CL = np.sqrt((2 * np.arange(_NUM_SPH) + 1) / (4 * np.pi))

_NSK = _NUM_SPH * _NUM_RAD
_Z42 = _Z.reshape(1, _NSK).astype(np.float32)
_K42 = (_NORMC * _CL[:, None]).reshape(1, _NSK).astype(np.float32)
_L42 = np.repeat(np.arange(_NUM_SPH), _NUM_RAD)
_LMASK = [(_L42 == l).reshape(1, _NSK) for l in range(_NUM_SPH)]


_SC_NC = 2
_SC_NS = 16
_NW = _SC_NC * _SC_NS


def _make_sc_gather(T):
    b_per_w = T // _NW
    mesh = plsc.VectorSubcoreMesh(core_axis_name="c", subcore_axis_name="s")

    @functools.partial(
        pl.kernel,
        mesh=mesh,
        out_type=jax.ShapeDtypeStruct((T,), jnp.float32),
        scratch_types=[
            pltpu.VMEM((b_per_w,), jnp.int32),
            pltpu.VMEM((b_per_w,), jnp.float32),
            pltpu.SemaphoreType.DMA,
        ],
    )
    def sc_gather(dist_hbm, idx_hbm, out_hbm, idx_v, rows_v, sem):
        wid = lax.axis_index("s") * _SC_NC + lax.axis_index("c")
        base = wid * b_per_w
        pltpu.sync_copy(idx_hbm.at[pl.ds(base, b_per_w)], idx_v)
        pltpu.async_copy(dist_hbm.at[idx_v], rows_v, sem).wait()
        pltpu.sync_copy(rows_v, out_hbm.at[pl.ds(base, b_per_w)])

    return sc_gather


_PACK = 5
_W = _PACK * _NSK
_BR = 1024

_TWO_OVER_PI = np.float32(0.63661975)
_PIO2_1 = np.float32(1.5707964)
_PIO2_2 = np.float32(-4.371139e-8)
_S1, _S2 = np.float32(-1.6666667e-1), np.float32(8.3333310e-3)
_S3, _S4 = np.float32(-1.9841271e-4), np.float32(2.7557314e-6)
_C1, _C2 = np.float32(-0.5), np.float32(4.1666668e-2)
_C3, _C4 = np.float32(-1.3888889e-3), np.float32(2.4801587e-5)


def _fast_sincos(x):
    kf = jnp.round(x * _TWO_OVER_PI)
    r = (x - kf * _PIO2_1) - kf * _PIO2_2
    z = r * r
    sp = r + r * z * (_S1 + z * (_S2 + z * _S3))
    cp = np.float32(1.0) + z * (_C1 + z * (_C2 + z * _C3))
    swap = (kf == np.float32(1.0)) | (kf == np.float32(3.0))
    s_val = jnp.where(swap, cp, sp)
    c_val = jnp.where(swap, sp, cp)
    s = jnp.where(kf >= np.float32(2.0), -s_val, s_val)
    c = jnp.where((kf == np.float32(1.0)) | (kf == np.float32(2.0)), -c_val, c_val)
    return s, c


def _fast_cos_small(x):
    m = x >= np.float32(0.78539816)
    r = jnp.where(m, (x - _PIO2_1) - _PIO2_2, x)
    z = r * r
    sp = r + r * z * (_S1 + z * (_S2 + z * _S3))
    cp = np.float32(1.0) + z * (_C1 + z * (_C2 + z * _C3))
    return jnp.where(m, -sp, cp)


def _tc_body(z_ref, k_ref, *refs):
    d_refs = refs[0:_PACK]
    a_refs = refs[_PACK : 2 * _PACK]
    o_ref = refs[2 * _PACK]
    zw = z_ref[...].reshape(1, _W)
    kw = k_ref[...].reshape(1, _W)
    col = lax.broadcasted_iota(jnp.int32, (1, _W), 1)
    lcol = (col % _NSK) // _NUM_RAD
    cgrp = col // _NSK

    def expand(vals_g):
        bs = [jnp.broadcast_to(v.reshape(_BR, 1), (_BR, _W)) for v in vals_g]

        def tree(lo, hi):
            if hi - lo == 1:
                return bs[lo]
            mid = (lo + hi) // 2
            return jnp.where(cgrp < mid, tree(lo, mid), tree(mid, hi))

        return tree(0, _PACK)

    d = expand([r[...] / np.float32(_CUTOFF) for r in d_refs])
    x = zw * d
    xs = jnp.maximum(x, np.float32(1e-12))
    s, c = _fast_sincos(xs)
    inv = np.float32(1.0) / xs
    j0 = s * inv
    j1 = s / (xs * xs) - c / xs

    ctb = expand([_fast_cos_small(r[...]) for r in a_refs])

    prod = jnp.where(lcol == 0, j0, j1 * ctb)
    jm, jc_ = j0, j1
    jm, jc_ = jc_, np.float32(3) * inv * jc_ - jm
    pm, pc = ctb, np.float32(1.5) * ctb * ctb - np.float32(0.5)
    prod = jnp.where(lcol == 2, jc_ * pc, prod)
    for i in range(2, _NUM_SPH - 1):
        jm, jc_ = jc_, np.float32(2 * i + 1) * inv * jc_ - jm
        pm, pc = pc, (np.float32(2 * i + 1) * ctb * pc - np.float32(i) * pm) * np.float32(1.0 / (i + 1))
        prod = jnp.where(lcol == i + 1, jc_ * pc, prod)

    outw = kw * prod
    for g in range(_PACK):
        o_ref[g, :, :] = outw[:, g * _NSK : (g + 1) * _NSK]


_NCHUNK = 2


def _tc_compute(d_g, angle):
    T = d_g.shape[0]
    TC = T // _NCHUNK
    R = TC // _PACK
    assert R % _BR == 0, (T, _PACK, _BR)
    nb = R // _BR
    ztile = np.tile(_Z42.reshape(-1), _PACK)
    ktile = np.tile(_K42.reshape(-1), _PACK)
    zc = jnp.asarray(ztile)
    kc = jnp.asarray(ktile)

    pieces = []
    for ch in range(_NCHUNK):
        base = ch * (TC // _BR)

        def mk_spec(g, base=base):
            return pl.BlockSpec((_BR,), lambda i, g=g, base=base: (base + g * nb + i,))

        out = pl.pallas_call(
            _tc_body,
            grid=(nb,),
            in_specs=[
                pl.BlockSpec((_W,), lambda i: (0,)),
                pl.BlockSpec((_W,), lambda i: (0,)),
            ] + [mk_spec(g) for g in range(_PACK)] + [mk_spec(g) for g in range(_PACK)],
            out_specs=pl.BlockSpec((_PACK, _BR, _NSK), lambda i: (0, i, 0)),
            out_shape=jax.ShapeDtypeStruct((_PACK, R, _NSK), jnp.float32),
        )(zc, kc, *([d_g] * _PACK), *([angle] * _PACK))
        pieces.append(out.reshape(TC, _NSK))
    if _NCHUNK == 1:
        return pieces[0]
    return jnp.concatenate(pieces, axis=0)


@jax.jit
def kernel(dist, angle, idx_kj):
    T = idx_kj.shape[0]
    d_g = _make_sc_gather(T)(dist, idx_kj.astype(jnp.int32))
    return _tc_compute(d_g, angle)


# speedup vs baseline: 1.1027x; 1.1027x over previous
import functools

import numpy as np
import jax
import jax.numpy as jnp
from jax import lax
from jax.experimental import pallas as pl
from jax.experimental.pallas import tpu as pltpu
from jax.experimental.pallas import tpu_sc as plsc

_NUM_SPH = 7
_NUM_RAD = 6
_CUTOFF = 5.0


def _sph_jl_np(l, x):
    x = np.asarray(x, dtype=np.float64)
    j0 = np.sin(x) / x
    if l == 0:
        return j0
    j1 = np.sin(x) / x**2 - np.cos(x) / x
    if l == 1:
        return j1
    jm, jc = j0, j1
    for i in range(1, l):
        jm, jc = jc, (2 * i + 1) / x * jc - jm
    return jc


def _sph_zeros(n, k):
    m = n + k
    zeros = [np.arange(1, m + 1) * np.pi]
    for l in range(1, n):
        prev = zeros[-1]
        cur = []
        for j in range(len(prev) - 1):
            a, b = float(prev[j]), float(prev[j + 1])
            fa = float(_sph_jl_np(l, a))
            for _ in range(100):
                c = 0.5 * (a + b)
                fc = float(_sph_jl_np(l, c))
                if fa * fc <= 0.0:
                    b = c
                else:
                    a, fa = c, fc
            cur.append(0.5 * (a + b))
        zeros.append(np.asarray(cur))
    return np.stack([z[:k] for z in zeros], axis=0)


_Z = _sph_zeros(_NUM_SPH, _NUM_RAD)
_NORMC = np.zeros((_NUM_SPH, _NUM_RAD))
for _l in range(_NUM_SPH):
    for _j in range(_NUM_RAD):
        _NORMC[_l, _j] = 1.0 / np.sqrt(0.5 * _sph_jl_np(_l + 1, _Z[_l, _j]) ** 2)
_CL = np.sqrt((2 * np.arange(_NUM_SPH) + 1) / (4 * np.pi))

_NSK = _NUM_SPH * _NUM_RAD
_Z42 = _Z.reshape(1, _NSK).astype(np.float32)

_A_L = np.ones(_NUM_SPH)
for _i in range(1, _NUM_SPH - 1):
    _A_L[_i + 1] = (2 * _i + 1) * _A_L[_i] / (_i + 1)
_B_I = [float(i * i) / (4 * i * i - 1) for i in range(_NUM_SPH)]
_K42 = (_NORMC * _CL[:, None] * _A_L[:, None]).reshape(1, _NSK).astype(np.float32)


_SC_NC = 2
_SC_NS = 16
_NW = _SC_NC * _SC_NS


def _make_sc_gather(T):
    b_per_w = T // _NW
    mesh = plsc.VectorSubcoreMesh(core_axis_name="c", subcore_axis_name="s")

    @functools.partial(
        pl.kernel,
        mesh=mesh,
        out_type=jax.ShapeDtypeStruct((T,), jnp.float32),
        scratch_types=[
            pltpu.VMEM((b_per_w,), jnp.int32),
            pltpu.VMEM((b_per_w,), jnp.float32),
            pltpu.SemaphoreType.DMA,
        ],
    )
    def sc_gather(dist_hbm, idx_hbm, out_hbm, idx_v, rows_v, sem):
        wid = lax.axis_index("s") * _SC_NC + lax.axis_index("c")
        base = wid * b_per_w
        pltpu.sync_copy(idx_hbm.at[pl.ds(base, b_per_w)], idx_v)
        pltpu.async_copy(dist_hbm.at[idx_v], rows_v, sem).wait()
        pltpu.sync_copy(rows_v, out_hbm.at[pl.ds(base, b_per_w)])

    return sc_gather


_PACK = 5
_W = _PACK * _NSK
_BR = 2048

_TWO_OVER_PI = np.float32(0.63661975)
_PIO2_1 = np.float32(1.5707964)
_PIO2_2 = np.float32(-4.371139e-8)
_S1, _S2 = np.float32(-1.6666667e-1), np.float32(8.3333310e-3)
_S3, _S4 = np.float32(-1.9841271e-4), np.float32(2.7557314e-6)
_C1, _C2 = np.float32(-0.5), np.float32(4.1666668e-2)
_C3, _C4 = np.float32(-1.3888889e-3), np.float32(2.4801587e-5)


def _fast_sincos(x):
    kf = jnp.round(x * _TWO_OVER_PI)
    r = x - kf * _PIO2_1
    z = r * r
    sp = r + r * z * (_S1 + z * (_S2 + z * _S3))
    cp = np.float32(1.0) + z * (_C1 + z * (_C2 + z * _C3))
    swap = (kf == np.float32(1.0)) | (kf == np.float32(3.0))
    s_val = jnp.where(swap, cp, sp)
    c_val = jnp.where(swap, sp, cp)
    s = jnp.where(kf >= np.float32(2.0), -s_val, s_val)
    c = jnp.where((kf == np.float32(1.0)) | (kf == np.float32(2.0)), -c_val, c_val)
    return s, c


def _fast_cos_small(x):
    m = x >= np.float32(0.78539816)
    r = jnp.where(m, (x - _PIO2_1) - _PIO2_2, x)
    z = r * r
    sp = r + r * z * (_S1 + z * (_S2 + z * _S3))
    cp = np.float32(1.0) + z * (_C1 + z * (_C2 + z * _C3))
    return jnp.where(m, -sp, cp)


def _tc_body(z_ref, k_ref, *refs):
    d_refs = refs[0:_PACK]
    a_refs = refs[_PACK : 2 * _PACK]
    o_ref = refs[2 * _PACK]
    zw = z_ref[...].reshape(1, _W)
    kw = k_ref[...].reshape(1, _W)
    col = lax.broadcasted_iota(jnp.int32, (1, _W), 1)
    lcol = (col % _NSK) // _NUM_RAD
    cgrp = col // _NSK

    def expand(vals_g):
        bs = [jnp.broadcast_to(v.reshape(_BR, 1), (_BR, _W)) for v in vals_g]

        def tree(lo, hi):
            if hi - lo == 1:
                return bs[lo]
            mid = (lo + hi) // 2
            return jnp.where(cgrp < mid, tree(lo, mid), tree(mid, hi))

        return tree(0, _PACK)

    d = expand([r[...] / np.float32(_CUTOFF) for r in d_refs])
    x = zw * d
    xs = jnp.maximum(x, np.float32(1e-12))
    s, c = _fast_sincos(xs)
    inv = np.float32(1.0) / xs
    j0 = s * inv
    j1 = s / (xs * xs) - c / xs

    ctb = expand([_fast_cos_small(r[...]) for r in a_refs])

    prod = jnp.where(lcol == 0, j0, j1 * ctb)
    jm, jc_ = j0, j1
    jm, jc_ = jc_, np.float32(3) * inv * jc_ - jm
    pm, pc = ctb, ctb * ctb - np.float32(_B_I[1])
    prod = jnp.where(lcol == 2, jc_ * pc, prod)
    for i in range(2, _NUM_SPH - 1):
        jm, jc_ = jc_, np.float32(2 * i + 1) * inv * jc_ - jm
        pm, pc = pc, ctb * pc - np.float32(_B_I[i]) * pm
        prod = jnp.where(lcol == i + 1, jc_ * pc, prod)

    outw = kw * prod
    for g in range(_PACK):
        o_ref[g, :, :] = outw[:, g * _NSK : (g + 1) * _NSK]


_NCHUNK = 1


def _tc_compute(d_g, angle):
    T = d_g.shape[0]
    TC = T // _NCHUNK
    R = TC // _PACK
    assert R % _BR == 0, (T, _PACK, _BR)
    nb = R // _BR
    ztile = np.tile(_Z42.reshape(-1), _PACK)
    ktile = np.tile(_K42.reshape(-1), _PACK)
    zc = jnp.asarray(ztile)
    kc = jnp.asarray(ktile)

    pieces = []
    for ch in range(_NCHUNK):
        base = ch * (TC // _BR)

        def mk_spec(g, base=base):
            return pl.BlockSpec((_BR,), lambda i, g=g, base=base: (base + g * nb + i,))

        out = pl.pallas_call(
            _tc_body,
            grid=(nb,),
            in_specs=[
                pl.BlockSpec((_W,), lambda i: (0,)),
                pl.BlockSpec((_W,), lambda i: (0,)),
            ] + [mk_spec(g) for g in range(_PACK)] + [mk_spec(g) for g in range(_PACK)],
            out_specs=pl.BlockSpec((_PACK, _BR, _NSK), lambda i: (0, i, 0)),
            out_shape=jax.ShapeDtypeStruct((_PACK, R, _NSK), jnp.float32),
        )(zc, kc, *([d_g] * _PACK), *([angle] * _PACK))
        pieces.append(out.reshape(TC, _NSK))
    if _NCHUNK == 1:
        return pieces[0]
    return jnp.concatenate(pieces, axis=0)


@jax.jit
def kernel(dist, angle, idx_kj):
    T = idx_kj.shape[0]
    d_g = _make_sc_gather(T)(dist, idx_kj.astype(jnp.int32))
    return _tc_compute(d_g, angle)
